# baseline (speedup 1.0000x reference)
import functools

import jax
import jax.numpy as jnp
from jax import lax
from jax.experimental import pallas as pl
from jax.experimental.pallas import tpu as pltpu
from jax.experimental.pallas import tpu_sc as plsc


def _sc_adjacency(edge_index, n):
    e_total = edge_index.shape[1]
    info = plsc.get_sparse_core_info()
    nc, ns, L = info.num_cores, info.num_subcores, info.num_lanes
    nw = nc * ns
    rows_w = n // nw
    wseg = rows_w * n
    wbuf = wseg + L
    mesh = plsc.VectorSubcoreMesh(core_axis_name="c", subcore_axis_name="s")

    @functools.partial(
        pl.kernel,
        mesh=mesh,
        out_type=jax.ShapeDtypeStruct((n * n,), jnp.float32),
        scratch_types=[
            pltpu.VMEM((e_total,), jnp.int32),
            pltpu.VMEM((e_total,), jnp.int32),
            pltpu.VMEM((wbuf,), jnp.float32),
        ],
        compiler_params=pltpu.CompilerParams(needs_layout_passes=False),
    )
    def adj(e_hbm, out_hbm, src_v, dst_v, aw_v):
        c = lax.axis_index("c")
        s = lax.axis_index("s")
        w = c * ns + s
        lo = w * rows_w
        zero16 = jnp.zeros((L,), jnp.float32)
        one16 = jnp.ones((L,), jnp.float32)

        zunroll = 8

        def zbody(i, _):
            for u in range(zunroll):
                aw_v[pl.ds((i * zunroll + u) * L, L)] = zero16
            return 0

        lax.fori_loop(0, wbuf // (L * zunroll), zbody, 0)
        pltpu.sync_copy(e_hbm.at[0], src_v)
        pltpu.sync_copy(e_hbm.at[1], dst_v)

        eunroll = 8

        def ebody(i, _):
            for u in range(eunroll):
                off = (i * eunroll + u) * L
                d16 = dst_v[pl.ds(off, L)]
                s16 = src_v[pl.ds(off, L)]
                inr = (d16 >= lo) & (d16 < lo + rows_w)
                lin = jnp.where(inr, (d16 - lo) * n + s16, wseg)
                plsc.addupdate_scatter(aw_v, [lin], one16)
            return 0

        lax.fori_loop(0, e_total // (L * eunroll), ebody, 0)
        pltpu.sync_copy(aw_v.at[pl.ds(0, wseg)],
                        out_hbm.at[pl.ds(w * wseg, wseg)])

    return adj(edge_index).reshape(n, n)


def _gin_body(e1_ref, e2_ref, a_ref, x_ref, k1_ref, k2_ref, k3_ref,
              b1a_ref, b2a_ref, b1b_ref, w2b_ref, b2b_ref, out_ref):
    i = pl.program_id(0)
    n = a_ref.shape[0]
    d_out = w2b_ref.shape[1]
    wide = k2_ref.shape[0]
    f32 = jnp.float32
    bf16 = jnp.bfloat16

    a = a_ref[...].astype(bf16)
    e1 = 1.0 + e1_ref[0, 0]
    e2 = 1.0 + e2_ref[0, 0]

    xi = x_ref[...].astype(bf16)
    h1 = jnp.dot(a, xi, preferred_element_type=f32) + e1 * xi
    t1 = jax.nn.relu(
        jnp.dot(h1.astype(bf16), k1_ref[...], preferred_element_type=f32)
        + b1a_ref[...])
    x1 = jax.nn.relu(
        jnp.dot(t1.astype(bf16), k2_ref[...], preferred_element_type=f32)
        + b2a_ref[...])

    x1b = x1.astype(bf16)
    h2 = jnp.dot(a, x1b, preferred_element_type=f32) + e2 * x1
    t2 = jax.nn.relu(
        jnp.dot(h2.astype(bf16), k3_ref[...], preferred_element_type=f32)
        + b1b_ref[...])

    d_h = w2b_ref.shape[0]
    mb = wide // d_h
    rs = [jnp.sum(t2[:, f * mb:(f + 1) * mb], axis=1, keepdims=True)
          for f in range(d_h)]
    pe = sum(rs[f] * w2b_ref[f:f + 1, :] for f in range(d_h))

    @pl.when(i == 0)
    def _():
        out_ref[...] = float(n) * jnp.broadcast_to(b2b_ref[...], (n, d_out))

    out_ref[...] += pe


def kernel(W_list, edge_index, w1a, b1a, w2a, b2a, eps1, w1b, b1b, w2b, b2b,
           eps2):
    n_graphs, n_max, n_nodes_dim, d_in = (W_list.shape[0], W_list.shape[1],
                                          W_list.shape[2], W_list.shape[3])
    n = n_graphs * n_max
    m = n_nodes_dim
    d_h = w1a.shape[1]
    d_out = w2b.shape[1]
    e_total = edge_index.shape[1]

    x0f = W_list.reshape(n, m * d_in)

    adj = _sc_adjacency(edge_index, n)

    mb = 32
    grid = m // mb
    bf16 = jnp.bfloat16
    eye = jnp.eye(mb, dtype=jnp.float32)
    k1 = jnp.einsum('jk,df->jdfk', eye, w1a).reshape(
        mb * d_in, mb * d_h).astype(bf16)
    k2 = jnp.einsum('jk,fg->fjgk', eye, w2a).reshape(
        mb * d_h, mb * d_h).astype(bf16)
    k3 = jnp.einsum('jk,fg->fjgk', eye, w1b).reshape(
        mb * d_h, mb * d_h).astype(bf16)
    b1a_r = jnp.repeat(b1a, mb)[None, :]
    b2a_r = jnp.repeat(b2a, mb)[None, :]
    b1b_r = jnp.repeat(b1b, mb)[None, :]
    smem = pltpu.SMEM
    full = lambda i: (0, 0)
    pe = pl.pallas_call(
        _gin_body,
        grid=(grid,),
        in_specs=[
            pl.BlockSpec(memory_space=smem),
            pl.BlockSpec(memory_space=smem),
            pl.BlockSpec((n, n), full),
            pl.BlockSpec((n, mb * d_in), lambda i: (0, i)),
            pl.BlockSpec((mb * d_in, mb * d_h), full),
            pl.BlockSpec((mb * d_h, mb * d_h), full),
            pl.BlockSpec((mb * d_h, mb * d_h), full),
            pl.BlockSpec((1, mb * d_h), full),
            pl.BlockSpec((1, mb * d_h), full),
            pl.BlockSpec((1, mb * d_h), full),
            pl.BlockSpec((d_h, d_out), full),
            pl.BlockSpec((1, d_out), full),
        ],
        out_specs=pl.BlockSpec((n, d_out), full),
        out_shape=jax.ShapeDtypeStruct((n, d_out), jnp.float32),
    )(
        eps1.reshape(1, 1), eps2.reshape(1, 1),
        adj, x0f, k1, k2, k3, b1a_r, b2a_r, b1b_r,
        w2b, b2b.reshape(1, d_out),
    )
    return pe


# speedup vs baseline: 1.0010x; 1.0010x over previous
import functools

import jax
import jax.numpy as jnp
from jax import lax
from jax.experimental import pallas as pl
from jax.experimental.pallas import tpu as pltpu
from jax.experimental.pallas import tpu_sc as plsc


def _sc_adjacency(edge_index, n):
    e_total = edge_index.shape[1]
    info = plsc.get_sparse_core_info()
    nc, ns, L = info.num_cores, info.num_subcores, info.num_lanes
    nw = nc * ns
    rows_w = n // nw
    wseg = rows_w * n
    wbuf = wseg + L
    mesh = plsc.VectorSubcoreMesh(core_axis_name="c", subcore_axis_name="s")

    @functools.partial(
        pl.kernel,
        mesh=mesh,
        out_type=jax.ShapeDtypeStruct((n * n,), jnp.float32),
        scratch_types=[
            pltpu.VMEM((e_total,), jnp.int32),
            pltpu.VMEM((e_total,), jnp.int32),
            pltpu.VMEM((wbuf,), jnp.float32),
        ],
        compiler_params=pltpu.CompilerParams(needs_layout_passes=False),
    )
    def adj(e_hbm, out_hbm, src_v, dst_v, aw_v):
        c = lax.axis_index("c")
        s = lax.axis_index("s")
        w = c * ns + s
        lo = w * rows_w
        zero16 = jnp.zeros((L,), jnp.float32)
        one16 = jnp.ones((L,), jnp.float32)

        zunroll = 8

        def zbody(i, _):
            for u in range(zunroll):
                aw_v[pl.ds((i * zunroll + u) * L, L)] = zero16
            return 0

        lax.fori_loop(0, wbuf // (L * zunroll), zbody, 0)
        pltpu.sync_copy(e_hbm.at[0], src_v)
        pltpu.sync_copy(e_hbm.at[1], dst_v)

        eunroll = 8

        def ebody(i, _):
            for u in range(eunroll):
                off = (i * eunroll + u) * L
                d16 = dst_v[pl.ds(off, L)]
                s16 = src_v[pl.ds(off, L)]
                inr = (d16 >= lo) & (d16 < lo + rows_w)
                lin = jnp.where(inr, (d16 - lo) * n + s16, wseg)
                plsc.addupdate_scatter(aw_v, [lin], one16)
            return 0

        lax.fori_loop(0, e_total // (L * eunroll), ebody, 0)
        pltpu.sync_copy(aw_v.at[pl.ds(0, wseg)],
                        out_hbm.at[pl.ds(w * wseg, wseg)])

    return adj(edge_index).reshape(n, n)


def _gin_body(e1_ref, e2_ref, a_ref, x_ref, k1_ref, k2_ref, k3_ref,
              b1a_ref, b2a_ref, b1b_ref, w2b_ref, b2b_ref, out_ref,
              abf_ref):
    i = pl.program_id(0)
    n = a_ref.shape[0]
    d_out = w2b_ref.shape[1]
    wide = k2_ref.shape[0]
    f32 = jnp.float32
    bf16 = jnp.bfloat16

    @pl.when(i == 0)
    def _():
        abf_ref[...] = a_ref[...].astype(bf16)

    a = abf_ref[...]
    e1 = 1.0 + e1_ref[0, 0]
    e2 = 1.0 + e2_ref[0, 0]

    xi = x_ref[...].astype(bf16)
    h1 = jnp.dot(a, xi, preferred_element_type=f32) + e1 * xi
    t1 = jax.nn.relu(
        jnp.dot(h1.astype(bf16), k1_ref[...], preferred_element_type=f32)
        + b1a_ref[...])
    x1 = jax.nn.relu(
        jnp.dot(t1.astype(bf16), k2_ref[...], preferred_element_type=f32)
        + b2a_ref[...])

    x1b = x1.astype(bf16)
    h2 = jnp.dot(a, x1b, preferred_element_type=f32) + e2 * x1
    t2 = jax.nn.relu(
        jnp.dot(h2.astype(bf16), k3_ref[...], preferred_element_type=f32)
        + b1b_ref[...])

    d_h = w2b_ref.shape[0]
    mb = wide // d_h
    rs = [jnp.sum(t2[:, f * mb:(f + 1) * mb], axis=1, keepdims=True)
          for f in range(d_h)]
    pe = sum(rs[f] * w2b_ref[f:f + 1, :] for f in range(d_h))

    @pl.when(i == 0)
    def _():
        out_ref[...] = float(n) * jnp.broadcast_to(b2b_ref[...], (n, d_out))

    out_ref[...] += pe


def kernel(W_list, edge_index, w1a, b1a, w2a, b2a, eps1, w1b, b1b, w2b, b2b,
           eps2):
    n_graphs, n_max, n_nodes_dim, d_in = (W_list.shape[0], W_list.shape[1],
                                          W_list.shape[2], W_list.shape[3])
    n = n_graphs * n_max
    m = n_nodes_dim
    d_h = w1a.shape[1]
    d_out = w2b.shape[1]
    e_total = edge_index.shape[1]

    x0f = W_list.reshape(n, m * d_in)

    adj = _sc_adjacency(edge_index, n)

    mb = 32
    grid = m // mb
    bf16 = jnp.bfloat16
    eye = jnp.eye(mb, dtype=jnp.float32)
    k1 = jnp.einsum('jk,df->jdfk', eye, w1a).reshape(
        mb * d_in, mb * d_h).astype(bf16)
    k2 = jnp.einsum('jk,fg->fjgk', eye, w2a).reshape(
        mb * d_h, mb * d_h).astype(bf16)
    k3 = jnp.einsum('jk,fg->fjgk', eye, w1b).reshape(
        mb * d_h, mb * d_h).astype(bf16)
    b1a_r = jnp.repeat(b1a, mb)[None, :]
    b2a_r = jnp.repeat(b2a, mb)[None, :]
    b1b_r = jnp.repeat(b1b, mb)[None, :]
    smem = pltpu.SMEM
    full = lambda i: (0, 0)
    pe = pl.pallas_call(
        _gin_body,
        grid=(grid,),
        in_specs=[
            pl.BlockSpec(memory_space=smem),
            pl.BlockSpec(memory_space=smem),
            pl.BlockSpec((n, n), full),
            pl.BlockSpec((n, mb * d_in), lambda i: (0, i)),
            pl.BlockSpec((mb * d_in, mb * d_h), full),
            pl.BlockSpec((mb * d_h, mb * d_h), full),
            pl.BlockSpec((mb * d_h, mb * d_h), full),
            pl.BlockSpec((1, mb * d_h), full),
            pl.BlockSpec((1, mb * d_h), full),
            pl.BlockSpec((1, mb * d_h), full),
            pl.BlockSpec((d_h, d_out), full),
            pl.BlockSpec((1, d_out), full),
        ],
        out_specs=pl.BlockSpec((n, d_out), full),
        out_shape=jax.ShapeDtypeStruct((n, d_out), jnp.float32),
        scratch_shapes=[pltpu.VMEM((n, n), bf16)],
    )(
        eps1.reshape(1, 1), eps2.reshape(1, 1),
        adj, x0f, k1, k2, k3, b1a_r, b2a_r, b1b_r,
        w2b, b2b.reshape(1, d_out),
    )
    return pe


# speedup vs baseline: 1.5394x; 1.5379x over previous
import functools

import jax
import jax.numpy as jnp
from jax import lax
from jax.experimental import pallas as pl
from jax.experimental.pallas import tpu as pltpu
from jax.experimental.pallas import tpu_sc as plsc


def _sc_adjacency(edge_index, n):
    e_total = edge_index.shape[1]
    info = plsc.get_sparse_core_info()
    nc, ns, L = info.num_cores, info.num_subcores, info.num_lanes
    nw = nc * ns
    rows_w = n // nw
    wseg = rows_w * n
    wbuf = wseg + L
    mesh = plsc.VectorSubcoreMesh(core_axis_name="c", subcore_axis_name="s")

    @functools.partial(
        pl.kernel,
        mesh=mesh,
        out_type=jax.ShapeDtypeStruct((n * n,), jnp.float32),
        scratch_types=[
            pltpu.VMEM((e_total,), jnp.int32),
            pltpu.VMEM((e_total,), jnp.int32),
            pltpu.VMEM((wbuf,), jnp.float32),
        ],
        compiler_params=pltpu.CompilerParams(needs_layout_passes=False),
    )
    def adj(e_hbm, out_hbm, src_v, dst_v, aw_v):
        c = lax.axis_index("c")
        s = lax.axis_index("s")
        w = c * ns + s
        lo = w * rows_w
        zero16 = jnp.zeros((L,), jnp.float32)
        one16 = jnp.ones((L,), jnp.float32)

        zunroll = 8

        def zbody(i, _):
            for u in range(zunroll):
                aw_v[pl.ds((i * zunroll + u) * L, L)] = zero16
            return 0

        lax.fori_loop(0, wbuf // (L * zunroll), zbody, 0)
        pltpu.sync_copy(e_hbm.at[0], src_v)
        pltpu.sync_copy(e_hbm.at[1], dst_v)

        eunroll = 8

        def ebody(i, _):
            for u in range(eunroll):
                off = (i * eunroll + u) * L
                d16 = dst_v[pl.ds(off, L)]
                s16 = src_v[pl.ds(off, L)]
                inr = (d16 >= lo) & (d16 < lo + rows_w)
                lin = jnp.where(inr, (d16 - lo) * n + s16, wseg)
                plsc.addupdate_scatter(aw_v, [lin], one16)
            return 0

        lax.fori_loop(0, e_total // (L * eunroll), ebody, 0)
        pltpu.sync_copy(aw_v.at[pl.ds(0, wseg)],
                        out_hbm.at[pl.ds(w * wseg, wseg)])

    return adj(edge_index).reshape(n, n)


def _gin_body(e1_ref, e2_ref, w1a_ref, w2a_ref, w1b_ref, b1a_ref, b2a_ref,
              b1b_ref, a_ref, x_ref, w2b_ref, b2b_ref, out_ref):
    i = pl.program_id(0)
    n = a_ref.shape[0]
    d_in = x_ref.shape[0]
    d_h = w1a_ref.shape[1]
    d_out = w2b_ref.shape[1]
    mb = x_ref.shape[2]
    f32 = jnp.float32

    a = a_ref[...].astype(jnp.bfloat16)
    e1 = 1.0 + e1_ref[0, 0]
    e2 = 1.0 + e2_ref[0, 0]

    xs = [x_ref[d] for d in range(d_in)]
    xcat = jnp.concatenate(xs, axis=1)
    hcat = jnp.dot(a, xcat, preferred_element_type=f32)
    hs = [hcat[:, d * mb:(d + 1) * mb] + e1 * xs[d] for d in range(d_in)]

    t1 = [
        jax.nn.relu(
            sum(hs[d] * w1a_ref[d, f] for d in range(d_in)) + b1a_ref[0, f])
        for f in range(d_h)
    ]
    x1 = [
        jax.nn.relu(
            sum(t1[f] * w2a_ref[f, g] for f in range(d_h)) + b2a_ref[0, g])
        for g in range(d_h)
    ]

    x1cat = jnp.concatenate(x1, axis=1).astype(jnp.bfloat16)
    h2cat = jnp.dot(a, x1cat, preferred_element_type=f32)
    hs2 = [h2cat[:, g * mb:(g + 1) * mb] + e2 * x1[g] for g in range(d_h)]

    t2 = [
        jax.nn.relu(
            sum(hs2[g] * w1b_ref[g, f] for g in range(d_h)) + b1b_ref[0, f])
        for f in range(d_h)
    ]

    rs = [jnp.sum(t2[f], axis=1, keepdims=True) for f in range(d_h)]
    pe = sum(rs[f] * w2b_ref[f:f + 1, :] for f in range(d_h))

    @pl.when(i == 0)
    def _():
        out_ref[...] = float(n) * jnp.broadcast_to(b2b_ref[...], (n, d_out))

    out_ref[...] += pe


def kernel(W_list, edge_index, w1a, b1a, w2a, b2a, eps1, w1b, b1b, w2b, b2b,
           eps2):
    n_graphs, n_max, n_nodes_dim, d_in = (W_list.shape[0], W_list.shape[1],
                                          W_list.shape[2], W_list.shape[3])
    n = n_graphs * n_max
    m = n_nodes_dim
    d_h = w1a.shape[1]
    d_out = w2b.shape[1]
    e_total = edge_index.shape[1]

    x0p = W_list.reshape(n, m, d_in).astype(jnp.bfloat16).transpose(2, 0, 1)

    adj = _sc_adjacency(edge_index, n)

    mb = 256
    grid = m // mb
    smem = pltpu.SMEM
    full = lambda i: (0, 0)
    pe = pl.pallas_call(
        _gin_body,
        grid=(grid,),
        in_specs=[
            pl.BlockSpec(memory_space=smem),
            pl.BlockSpec(memory_space=smem),
            pl.BlockSpec(memory_space=smem),
            pl.BlockSpec(memory_space=smem),
            pl.BlockSpec(memory_space=smem),
            pl.BlockSpec(memory_space=smem),
            pl.BlockSpec(memory_space=smem),
            pl.BlockSpec(memory_space=smem),
            pl.BlockSpec((n, n), full),
            pl.BlockSpec((d_in, n, mb), lambda i: (0, 0, i)),
            pl.BlockSpec((d_h, d_out), full),
            pl.BlockSpec((1, d_out), full),
        ],
        out_specs=pl.BlockSpec((n, d_out), full),
        out_shape=jax.ShapeDtypeStruct((n, d_out), jnp.float32),
    )(
        eps1.reshape(1, 1), eps2.reshape(1, 1), w1a, w2a, w1b,
        b1a.reshape(1, d_h), b2a.reshape(1, d_h), b1b.reshape(1, d_h),
        adj, x0p, w2b, b2b.reshape(1, d_out),
    )
    return pe
